# baseline (speedup 1.0000x reference)
import functools

import jax
import jax.numpy as jnp
from jax import lax
from jax.experimental import pallas as pl
from jax.experimental.pallas import tpu as pltpu
from jax.experimental.pallas import tpu_sc as plsc

NC = 2
NS = 16
NW = NC * NS
L = 16

CHUNK = 128
IDX_GRP = 8

N = 10000
E = 160000
D = 256
DH = D // 2

N_PAD = 10240
ROWS_PER_TILE = N_PAD // NS
E_PAD = 163840
NCHUNK_T = E_PAD // (NS * CHUNK)
NCHUNK_W = E_PAD // (NW * CHUNK)
PAD_DST = 10016

NGRP_T = NCHUNK_T // IDX_GRP
NGRP_W = NCHUNK_W // IDX_GRP
WB = ROWS_PER_TILE // CHUNK

_MESH = dict(core_axis_name="c", subcore_axis_name="s")


def _sc_aggregate(xa, xb, src, dst, iota_rows):
  @functools.partial(
      pl.kernel,
      out_type=(
          jax.ShapeDtypeStruct((N_PAD, DH), jnp.float32),
          jax.ShapeDtypeStruct((N_PAD, DH), jnp.float32),
      ),
      mesh=plsc.VectorSubcoreMesh(**_MESH),
      scratch_types=[
          pltpu.VMEM((IDX_GRP, CHUNK), jnp.int32),
          pltpu.VMEM((IDX_GRP, CHUNK), jnp.int32),
          pltpu.VMEM((WB, CHUNK), jnp.int32),
          pltpu.VMEM((CHUNK, DH), jnp.float32),
          pltpu.VMEM((CHUNK, DH), jnp.float32),
          pltpu.SemaphoreType.DMA,
          pltpu.SemaphoreType.DMA,
          pltpu.SemaphoreType.DMA,
          pltpu.SemaphoreType.DMA,
          pltpu.VMEM_SHARED((N_PAD, DH), jnp.float32),
      ],
  )
  def agg_kernel(xa_hbm, xb_hbm, src_hbm, dst_hbm, iota_hbm,
                 suma_hbm, sumb_hbm,
                 src_v, dst_v, riox_v, rows_v, rows_w, sem0, sem1,
                 ssem0, ssem1, acc):
    c = lax.axis_index("c")
    s = lax.axis_index("s")
    row0 = s * ROWS_PER_TILE

    @pl.loop(0, CHUNK)
    def _zero_rows(i):
      for k in range(DH // L):
        rows_v[i, k * L:(k + 1) * L] = jnp.zeros((L,), jnp.float32)

    pltpu.sync_copy(iota_hbm.at[s], riox_v)

    for q in range(WB):
      pltpu.sync_copy(rows_v, acc.at[riox_v.at[q]])

    plsc.subcore_barrier()

    bufs = (rows_v, rows_w)
    gsems = (sem0, sem1)
    ssems = (ssem0, ssem1)

    def _pipeline(x_hbm):
      gd = pltpu.async_copy(x_hbm.at[src_v.at[0]], bufs[0], gsems[0])
      sd = [None, None]
      for j in range(IDX_GRP):
        b = j % 2
        gd.wait()
        sd[b] = pltpu.async_copy(bufs[b], acc.at[dst_v.at[j]],
                                 ssems[b], add=True)
        if j + 1 < IDX_GRP:
          nb = (j + 1) % 2
          if sd[nb] is not None:
            sd[nb].wait()
          gd = pltpu.async_copy(x_hbm.at[src_v.at[j + 1]], bufs[nb],
                                gsems[nb])
      sd[0].wait()
      sd[1].wait()

    @pl.loop(0, NGRP_T)
    def _grp(h):
      pltpu.sync_copy(src_hbm.at[s, h], src_v)
      pltpu.sync_copy(dst_hbm.at[s, h], dst_v)

      @pl.when(c == 0)
      def _():
        _pipeline(xa_hbm)

      @pl.when(c == 1)
      def _():
        _pipeline(xb_hbm)

    plsc.subcore_barrier()

    for q in range(WB):
      r = pl.multiple_of(row0 + q * CHUNK, CHUNK)
      pltpu.sync_copy(acc.at[riox_v.at[q]], rows_v)

      @pl.when(c == 0)
      def _():
        pltpu.sync_copy(rows_v, suma_hbm.at[pl.ds(r, CHUNK)])

      @pl.when(c == 1)
      def _():
        pltpu.sync_copy(rows_v, sumb_hbm.at[pl.ds(r, CHUNK)])

  return agg_kernel(xa, xb, src, dst, iota_rows)


def _sc_count(dst, iota_rows):
  @functools.partial(
      pl.kernel,
      out_type=(
          jax.ShapeDtypeStruct((N_PAD, CHUNK), jnp.float32),
          jax.ShapeDtypeStruct((N_PAD, CHUNK), jnp.float32),
      ),
      mesh=plsc.VectorSubcoreMesh(**_MESH),
      scratch_types=[
          pltpu.VMEM((IDX_GRP, CHUNK), jnp.int32),
          pltpu.VMEM((WB, CHUNK), jnp.int32),
          pltpu.VMEM((CHUNK, CHUNK), jnp.float32),
          pltpu.VMEM_SHARED((N_PAD, CHUNK), jnp.float32),
      ],
  )
  def cnt_kernel(dst_hbm, iota_hbm, cnt0_hbm, cnt1_hbm,
                 dst_v, riox_v, ones_v, cacc):
    c = lax.axis_index("c")
    s = lax.axis_index("s")
    wid = s * NC + c
    row0 = s * ROWS_PER_TILE

    @pl.loop(0, CHUNK)
    def _zero_rows(i):
      for k in range(CHUNK // L):
        ones_v[i, k * L:(k + 1) * L] = jnp.zeros((L,), jnp.float32)

    pltpu.sync_copy(iota_hbm.at[s], riox_v)
    for q in range(WB):
      pltpu.sync_copy(ones_v, cacc.at[riox_v.at[q]])

    @pl.loop(0, CHUNK)
    def _fill_ones(i):
      for k in range(CHUNK // L):
        ones_v[i, k * L:(k + 1) * L] = jnp.ones((L,), jnp.float32)

    plsc.subcore_barrier()

    @pl.loop(0, NGRP_W)
    def _grp(h):
      pltpu.sync_copy(dst_hbm.at[wid, h], dst_v)
      for j in range(IDX_GRP):
        pltpu.sync_copy(ones_v, cacc.at[dst_v.at[j]], add=True)

    plsc.subcore_barrier()

    for q in range(WB):
      r = pl.multiple_of(row0 + q * CHUNK, CHUNK)
      pltpu.sync_copy(cacc.at[riox_v.at[q]], ones_v)

      @pl.when(c == 0)
      def _():
        pltpu.sync_copy(ones_v, cnt0_hbm.at[pl.ds(r, CHUNK)])

      @pl.when(c == 1)
      def _():
        pltpu.sync_copy(ones_v, cnt1_hbm.at[pl.ds(r, CHUNK)])

  return cnt_kernel(dst, iota_rows)


BLK = 256


def _tc_body(x_ref, sa_ref, sb_ref, c0_ref, c1_ref, wagg_ref, w_ref, b_ref,
             out_ref):
  cnt = c0_ref[:, 0:1] + c1_ref[:, 0:1]
  inv = 1.0 / jnp.maximum(cnt, 1.0)
  mean = jnp.concatenate([sa_ref[...], sb_ref[...]], axis=1) * inv
  agg = jnp.dot(mean, wagg_ref[...], preferred_element_type=jnp.float32)
  h = (jnp.dot(x_ref[...], w_ref[0:D, :], preferred_element_type=jnp.float32)
       + jnp.dot(agg, w_ref[D:2 * D, :], preferred_element_type=jnp.float32))
  nrm = jnp.sqrt(jnp.sum(h * h, axis=1, keepdims=True))
  out_ref[...] = h / jnp.maximum(nrm, 1.0) + b_ref[...]


def _tc_dense(x_pad, suma, sumb, cnt0, cnt1, W_agg, W, b2d):
  grid = N_PAD // BLK
  return pl.pallas_call(
      _tc_body,
      grid=(grid,),
      in_specs=[
          pl.BlockSpec((BLK, D), lambda i: (i, 0)),
          pl.BlockSpec((BLK, DH), lambda i: (i, 0)),
          pl.BlockSpec((BLK, DH), lambda i: (i, 0)),
          pl.BlockSpec((BLK, CHUNK), lambda i: (i, 0)),
          pl.BlockSpec((BLK, CHUNK), lambda i: (i, 0)),
          pl.BlockSpec((D, D), lambda i: (0, 0)),
          pl.BlockSpec((2 * D, D), lambda i: (0, 0)),
          pl.BlockSpec((1, D), lambda i: (0, 0)),
      ],
      out_specs=pl.BlockSpec((BLK, D), lambda i: (i, 0)),
      out_shape=jax.ShapeDtypeStruct((N_PAD, D), jnp.float32),
      compiler_params=pltpu.CompilerParams(
          dimension_semantics=("arbitrary",)),
  )(x_pad, suma, sumb, cnt0, cnt1, W_agg, W, b2d)


def kernel(x, edge_index, W_agg, W, b):
  src = edge_index[0]
  dst = edge_index[1]
  pad_e = E_PAD - E
  src_p = jnp.concatenate([src, jnp.zeros((pad_e,), jnp.int32)])
  dst_p = jnp.concatenate([dst, jnp.full((pad_e,), PAD_DST, jnp.int32)])
  src_t = src_p.reshape(NS, NGRP_T, IDX_GRP, CHUNK)
  dst_t = dst_p.reshape(NS, NGRP_T, IDX_GRP, CHUNK)
  dst_w = dst_p.reshape(NW, NGRP_W, IDX_GRP, CHUNK)

  xa = x[:, :DH]
  xb = x[:, DH:]
  iota_rows = jnp.arange(N_PAD, dtype=jnp.int32).reshape(NS, WB, CHUNK)

  suma, sumb = _sc_aggregate(xa, xb, src_t, dst_t, iota_rows)
  cnt0, cnt1 = _sc_count(dst_w, iota_rows)

  x_pad = jnp.pad(x, ((0, N_PAD - N), (0, 0)))
  out = _tc_dense(x_pad, suma, sumb, cnt0, cnt1, W_agg, W, b.reshape(1, D))
  return out[:N]


# speedup vs baseline: 1.0371x; 1.0371x over previous
import functools

import jax
import jax.numpy as jnp
from jax import lax
from jax.experimental import pallas as pl
from jax.experimental.pallas import tpu as pltpu
from jax.experimental.pallas import tpu_sc as plsc

NC = 2
NS = 16
NW = NC * NS
L = 16

CHUNK = 128
IDX_GRP = 8

N = 10000
E = 160000
D = 256
DH = D // 2

N_PAD = 10240
ROWS_PER_TILE = N_PAD // NS
E_PAD = 163840
NCHUNK_T = E_PAD // (NS * CHUNK)
NCHUNK_W = E_PAD // (NW * CHUNK)
PAD_DST = 10016

NGRP_T = NCHUNK_T // IDX_GRP
NGRP_W = NCHUNK_W // IDX_GRP
WB = ROWS_PER_TILE // CHUNK

_MESH = dict(core_axis_name="c", subcore_axis_name="s")


def _sc_aggregate(xa, xb, src, dst, iota_rows):
  @functools.partial(
      pl.kernel,
      out_type=(
          jax.ShapeDtypeStruct((N_PAD, DH), jnp.float32),
          jax.ShapeDtypeStruct((N_PAD, DH), jnp.float32),
          jax.ShapeDtypeStruct((N_PAD, DH), jnp.float32),
          jax.ShapeDtypeStruct((N_PAD, DH), jnp.float32),
      ),
      mesh=plsc.VectorSubcoreMesh(**_MESH),
      scratch_types=[
          pltpu.VMEM((IDX_GRP, CHUNK), jnp.int32),
          pltpu.VMEM((IDX_GRP, CHUNK), jnp.int32),
          pltpu.VMEM((WB, CHUNK), jnp.int32),
          pltpu.VMEM((CHUNK, DH), jnp.float32),
          pltpu.VMEM((CHUNK, DH), jnp.float32),
          pltpu.SemaphoreType.DMA,
          pltpu.SemaphoreType.DMA,
          pltpu.SemaphoreType.DMA,
          pltpu.SemaphoreType.DMA,
          pltpu.VMEM_SHARED((N_PAD, DH), jnp.float32),
      ],
  )
  def agg_kernel(xa_hbm, xb_hbm, src_hbm, dst_hbm, iota_hbm,
                 suma_hbm, sumb_hbm, cnt0_hbm, cnt1_hbm,
                 src_v, dst_v, riox_v, rows_v, rows_w, sem0, sem1,
                 ssem0, ssem1, acc):
    c = lax.axis_index("c")
    s = lax.axis_index("s")
    row0 = s * ROWS_PER_TILE

    @pl.loop(0, CHUNK)
    def _zero_rows(i):
      for k in range(DH // L):
        rows_v[i, k * L:(k + 1) * L] = jnp.zeros((L,), jnp.float32)

    pltpu.sync_copy(iota_hbm.at[s], riox_v)

    for q in range(WB):
      pltpu.sync_copy(rows_v, acc.at[riox_v.at[q]])

    plsc.subcore_barrier()

    bufs = (rows_v, rows_w)
    gsems = (sem0, sem1)
    ssems = (ssem0, ssem1)

    def _pipeline(x_hbm):
      gd = pltpu.async_copy(x_hbm.at[src_v.at[0]], bufs[0], gsems[0])
      sd = [None, None]
      for j in range(IDX_GRP):
        b = j % 2
        gd.wait()
        sd[b] = pltpu.async_copy(bufs[b], acc.at[dst_v.at[j]],
                                 ssems[b], add=True)
        if j + 1 < IDX_GRP:
          nb = (j + 1) % 2
          if sd[nb] is not None:
            sd[nb].wait()
          gd = pltpu.async_copy(x_hbm.at[src_v.at[j + 1]], bufs[nb],
                                gsems[nb])
      sd[0].wait()
      sd[1].wait()

    @pl.loop(0, NGRP_T)
    def _grp(h):
      pltpu.sync_copy(src_hbm.at[s, h], src_v)
      pltpu.sync_copy(dst_hbm.at[s, h], dst_v)

      @pl.when(c == 0)
      def _():
        _pipeline(xa_hbm)

      @pl.when(c == 1)
      def _():
        _pipeline(xb_hbm)

    plsc.subcore_barrier()

    for q in range(WB):
      r = pl.multiple_of(row0 + q * CHUNK, CHUNK)
      pltpu.sync_copy(acc.at[riox_v.at[q]], rows_v)

      @pl.when(c == 0)
      def _():
        pltpu.sync_copy(rows_v, suma_hbm.at[pl.ds(r, CHUNK)])

      @pl.when(c == 1)
      def _():
        pltpu.sync_copy(rows_v, sumb_hbm.at[pl.ds(r, CHUNK)])

    @pl.loop(0, CHUNK)
    def _zero_rows2(i):
      for k in range(DH // L):
        rows_v[i, k * L:(k + 1) * L] = jnp.zeros((L,), jnp.float32)

    plsc.subcore_barrier()

    for q in range(WB):
      pltpu.sync_copy(rows_v, acc.at[riox_v.at[q]])

    @pl.loop(0, CHUNK)
    def _fill_ones(i):
      for k in range(DH // L):
        rows_v[i, k * L:(k + 1) * L] = jnp.ones((L,), jnp.float32)

    plsc.subcore_barrier()

    @pl.loop(0, NGRP_T // NC)
    def _grp_cnt(hh):
      pltpu.sync_copy(dst_hbm.at[s, c * (NGRP_T // NC) + hh], dst_v)
      sd = []
      for j in range(IDX_GRP):
        sd.append(pltpu.async_copy(rows_v, acc.at[dst_v.at[j]],
                                   ssems[j % 2], add=True))
      for d in sd:
        d.wait()

    plsc.subcore_barrier()

    for q in range(WB):
      r = pl.multiple_of(row0 + q * CHUNK, CHUNK)
      pltpu.sync_copy(acc.at[riox_v.at[q]], rows_w)

      @pl.when(c == 0)
      def _():
        pltpu.sync_copy(rows_w, cnt0_hbm.at[pl.ds(r, CHUNK)])

      @pl.when(c == 1)
      def _():
        pltpu.sync_copy(rows_w, cnt1_hbm.at[pl.ds(r, CHUNK)])

  return agg_kernel(xa, xb, src, dst, iota_rows)


BLK = 256


def _tc_body(x_ref, sa_ref, sb_ref, c0_ref, c1_ref, wagg_ref, w_ref, b_ref,
             out_ref):
  cnt = c0_ref[:, 0:1] + c1_ref[:, 0:1]
  inv = 1.0 / jnp.maximum(cnt, 1.0)
  mean = jnp.concatenate([sa_ref[...], sb_ref[...]], axis=1) * inv
  agg = jnp.dot(mean, wagg_ref[...], preferred_element_type=jnp.float32)
  h = (jnp.dot(x_ref[...], w_ref[0:D, :], preferred_element_type=jnp.float32)
       + jnp.dot(agg, w_ref[D:2 * D, :], preferred_element_type=jnp.float32))
  nrm = jnp.sqrt(jnp.sum(h * h, axis=1, keepdims=True))
  out_ref[...] = h / jnp.maximum(nrm, 1.0) + b_ref[...]


def _tc_dense(x_pad, suma, sumb, cnt0, cnt1, W_agg, W, b2d):
  grid = N_PAD // BLK
  return pl.pallas_call(
      _tc_body,
      grid=(grid,),
      in_specs=[
          pl.BlockSpec((BLK, D), lambda i: (i, 0)),
          pl.BlockSpec((BLK, DH), lambda i: (i, 0)),
          pl.BlockSpec((BLK, DH), lambda i: (i, 0)),
          pl.BlockSpec((BLK, DH), lambda i: (i, 0)),
          pl.BlockSpec((BLK, DH), lambda i: (i, 0)),
          pl.BlockSpec((D, D), lambda i: (0, 0)),
          pl.BlockSpec((2 * D, D), lambda i: (0, 0)),
          pl.BlockSpec((1, D), lambda i: (0, 0)),
      ],
      out_specs=pl.BlockSpec((BLK, D), lambda i: (i, 0)),
      out_shape=jax.ShapeDtypeStruct((N_PAD, D), jnp.float32),
      compiler_params=pltpu.CompilerParams(
          dimension_semantics=("arbitrary",)),
  )(x_pad, suma, sumb, cnt0, cnt1, W_agg, W, b2d)


def kernel(x, edge_index, W_agg, W, b):
  src = edge_index[0]
  dst = edge_index[1]
  pad_e = E_PAD - E
  src_p = jnp.concatenate([src, jnp.zeros((pad_e,), jnp.int32)])
  dst_p = jnp.concatenate([dst, jnp.full((pad_e,), PAD_DST, jnp.int32)])
  src_t = src_p.reshape(NS, NGRP_T, IDX_GRP, CHUNK)
  dst_t = dst_p.reshape(NS, NGRP_T, IDX_GRP, CHUNK)

  xa = x[:, :DH]
  xb = x[:, DH:]
  iota_rows = jnp.arange(N_PAD, dtype=jnp.int32).reshape(NS, WB, CHUNK)

  suma, sumb, cnt0, cnt1 = _sc_aggregate(xa, xb, src_t, dst_t, iota_rows)

  x_pad = jnp.pad(x, ((0, N_PAD - N), (0, 0)))
  out = _tc_dense(x_pad, suma, sumb, cnt0, cnt1, W_agg, W, b.reshape(1, D))
  return out[:N]
